# Initial kernel scaffold; baseline (speedup 1.0000x reference)
import functools

import jax
import jax.numpy as jnp
import numpy as np
from jax import lax
from jax.experimental import pallas as pl
from jax.experimental.pallas import tpu as pltpu
from jax.experimental.pallas import tpu_sc as plsc

ROI_LO_I, ROI_HI_I = 40, 400
M_DIPS = 6
MIN_AREA = 1e-05
W_AREA = 1.0
W_CENTROID = 1.0
W_DEPTH = 0.2
UNDERFILL_FACTOR = 2.0
B, L = 64, 2048
HALF = 10
WN = 2 * HALF + 1
NEG = float("-inf")

NSUB = 16
ROWS_PER = B // NSUB


def _log_taps():
    sigma = 2.0
    radius = int(max(1.0, 3.0 * sigma))
    x = np.arange(-radius, radius + 1, dtype=np.float32)
    s2 = np.float32(sigma * sigma)
    g = np.exp(-(x ** 2) / (2.0 * s2)).astype(np.float32)
    taps = ((x ** 2 - s2) / s2 ** 2 * g).astype(np.float32)
    taps = (taps - taps.mean()).astype(np.float32)
    return taps


_TAPS = _log_taps()
_PAD = 128
_LP = L + 2 * _PAD


def _detect_body(t_ref, centers_ref, valid_ref, pad_ref, pool_ref):
    t = t_ref[:]
    pad_ref[:] = jnp.zeros((B, _LP), jnp.float32)
    pad_ref[:, _PAD:_PAD + L] = t
    acc = jnp.zeros((B, L), jnp.float32)
    for k in range(_TAPS.shape[0]):
        acc = acc + float(_TAPS[k]) * pad_ref[:, _PAD - 6 + k:_PAD - 6 + k + L]
    col = lax.broadcasted_iota(jnp.int32, (B, L), 1)
    roi = ((col >= ROI_LO_I) & (col <= ROI_HI_I)).astype(jnp.float32)
    scores = -acc * roi

    pool_ref[:] = jnp.full((B, _LP), NEG, jnp.float32)
    pool_ref[:, _PAD:_PAD + L] = scores
    pooled = pool_ref[:, _PAD - 5:_PAD - 5 + L]
    for d in range(1, 11):
        pooled = jnp.maximum(pooled, pool_ref[:, _PAD - 5 + d:_PAD - 5 + d + L])

    mean = jnp.mean(scores, axis=1, keepdims=True)
    keep = (scores == pooled) & (scores > mean)
    masked = jnp.where(keep, scores, NEG)

    col16 = lax.broadcasted_iota(jnp.int32, (B, 16), 1)
    centers16 = jnp.zeros((B, 16), jnp.int32)
    valid16 = jnp.zeros((B, 16), jnp.float32)
    for tk in range(M_DIPS):
        m = jnp.max(masked, axis=1, keepdims=True)
        ismax = masked == m
        c = jnp.min(jnp.where(ismax, col, L), axis=1, keepdims=True)
        centers16 = jnp.where(col16 == tk, c, centers16)
        v = (m > NEG).astype(jnp.float32)
        valid16 = jnp.where(col16 == tk, v, valid16)
        masked = jnp.where(col == c, NEG, masked)
    centers_ref[:] = centers16
    valid_ref[:] = valid16


def _detect(target):
    return pl.pallas_call(
        _detect_body,
        out_shape=[
            jax.ShapeDtypeStruct((B, 16), jnp.int32),
            jax.ShapeDtypeStruct((B, 16), jnp.float32),
        ],
        scratch_shapes=[
            pltpu.VMEM((B, _LP), jnp.float32),
            pltpu.VMEM((B, _LP), jnp.float32),
        ],
    )(target)


def _sc_loss_body(tgt_hbm, pred_hbm, centers_hbm, valid_hbm, out_hbm,
                  trow, prow, crow, vrow, accst, redbuf, outbuf, shared):
    wid = lax.axis_index("s")
    acc_area = jnp.zeros((16,), jnp.float32)
    acc_cent = jnp.zeros((16,), jnp.float32)
    acc_dep = jnp.zeros((16,), jnp.float32)
    acc_cnt = jnp.zeros((16,), jnp.float32)
    for rlocal in range(ROWS_PER):
        row = wid * ROWS_PER + rlocal
        pltpu.sync_copy(tgt_hbm.at[row], trow)
        pltpu.sync_copy(pred_hbm.at[row], prow)
        pltpu.sync_copy(centers_hbm.at[row], crow)
        pltpu.sync_copy(valid_hbm.at[row], vrow)
        c = crow[:]
        vld = vrow[:]
        s = jnp.maximum(c - HALF, 0)
        e = jnp.minimum(c + HALF, L - 1)
        n = e - s
        nf = n.astype(jnp.float32)
        lam_s = 300.0 + 0.5 * s.astype(jnp.float32)
        lam_e = 300.0 + 0.5 * e.astype(jnp.float32)
        ts = plsc.load_gather(trow, [s])
        te = plsc.load_gather(trow, [e])
        ps = plsc.load_gather(prow, [s])
        pe = plsc.load_gather(prow, [e])
        dlam = lam_e - lam_s + 1e-6
        zero = jnp.zeros((16,), jnp.float32)
        area_t = zero
        area_p = zero
        ct_num = zero
        ct_den = zero
        cp_num = zero
        cp_den = zero
        dsum = zero
        prev_dt = zero
        prev_dp = zero
        prev_lseg = zero
        for j in range(WN):
            idx = jnp.minimum(s + j, e)
            lseg = 300.0 + 0.5 * idx.astype(jnp.float32)
            tt = (lseg - lam_s) / dlam
            cont_t = jnp.maximum((1.0 - tt) * ts + tt * te, 1e-6)
            cont_p = jnp.maximum((1.0 - tt) * ps + tt * pe, 1e-6)
            tv = plsc.load_gather(trow, [idx])
            pv = plsc.load_gather(prow, [idx])
            dt = jnp.clip(1.0 - jnp.clip(tv / cont_t, 0.0, 2.0), 0.0, 1.0)
            dp = jnp.clip(1.0 - jnp.clip(pv / cont_p, 0.0, 2.0), 0.0, 1.0)
            jf = float(j)
            pm = jf <= nf
            if j > 0:
                sm = (jf - 1.0) < nf
                dl = lseg - prev_lseg
                area_t = area_t + jnp.where(sm, (dt + prev_dt) * 0.5 * dl, 0.0)
                area_p = area_p + jnp.where(sm, (dp + prev_dp) * 0.5 * dl, 0.0)
            wt = dt + 1e-7
            wp = dp + 1e-7
            ct_num = ct_num + jnp.where(pm, lseg * wt, 0.0)
            ct_den = ct_den + jnp.where(pm, wt, 0.0)
            cp_num = cp_num + jnp.where(pm, lseg * wp, 0.0)
            cp_den = cp_den + jnp.where(pm, wp, 0.0)
            dsum = dsum + jnp.where(pm, jnp.abs(dp - dt), 0.0)
            prev_dt = dt
            prev_dp = dp
            prev_lseg = lseg
        inv_at = 1.0 / (area_t + 1e-7)
        rel_err = jnp.abs(area_p - area_t) * inv_at
        underfill = jnp.maximum(area_t - area_p, 0.0) * inv_at
        area_term = rel_err + (UNDERFILL_FACTOR - 1.0) * underfill
        centroid_term = jnp.abs(cp_num / cp_den - ct_num / ct_den)
        depth_term = dsum / (nf + 1.0)
        valid = (vld > 0.5) & (e > s) & jnp.logical_not(area_t < MIN_AREA)
        acc_area = acc_area + jnp.where(valid, area_term, 0.0)
        acc_cent = acc_cent + jnp.where(valid, centroid_term, 0.0)
        acc_dep = acc_dep + jnp.where(valid, depth_term, 0.0)
        acc_cnt = acc_cnt + jnp.where(valid, 1.0, 0.0)
    accst[0] = acc_area
    accst[1] = acc_cent
    accst[2] = acc_dep
    accst[3] = acc_cnt
    pltpu.sync_copy(accst, shared.at[wid])
    plsc.subcore_barrier()

    @pl.when(wid == 0)
    def _():
        pltpu.sync_copy(shared, redbuf)
        tot_a = jnp.zeros((16,), jnp.float32)
        tot_c = jnp.zeros((16,), jnp.float32)
        tot_d = jnp.zeros((16,), jnp.float32)
        tot_n = jnp.zeros((16,), jnp.float32)
        for w in range(NSUB):
            tot_a = tot_a + redbuf[w, 0]
            tot_c = tot_c + redbuf[w, 1]
            tot_d = tot_d + redbuf[w, 2]
            tot_n = tot_n + redbuf[w, 3]
        a = jnp.sum(tot_a)
        cc = jnp.sum(tot_c)
        dd = jnp.sum(tot_d)
        cnt = jnp.sum(tot_n)
        den = jnp.maximum(cnt, 1.0)
        loss = W_AREA * (a / den) + W_CENTROID * (cc / den) + W_DEPTH * (dd / den)
        loss = jnp.where(cnt > 0.0, loss, 0.0)
        outbuf[:] = jnp.full((16,), loss)
        pltpu.sync_copy(outbuf, out_hbm)


_sc_loss = functools.partial(
    pl.kernel,
    out_type=jax.ShapeDtypeStruct((16,), jnp.float32),
    mesh=plsc.VectorSubcoreMesh(core_axis_name="c", subcore_axis_name="s",
                                num_cores=1),
    scratch_types=[
        pltpu.VMEM((L,), jnp.float32),
        pltpu.VMEM((L,), jnp.float32),
        pltpu.VMEM((16,), jnp.int32),
        pltpu.VMEM((16,), jnp.float32),
        pltpu.VMEM((4, 16), jnp.float32),
        pltpu.VMEM((NSUB, 4, 16), jnp.float32),
        pltpu.VMEM((16,), jnp.float32),
        pltpu.VMEM_SHARED((NSUB, 4, 16), jnp.float32),
    ],
)(_sc_loss_body)


def kernel(prediction, target, lam_nm):
    del lam_nm
    pred = prediction.astype(jnp.float32)
    tgt = target.astype(jnp.float32)
    centers, valid = _detect(tgt)
    out = _sc_loss(tgt, pred, centers, valid)
    return out[0]


# speedup vs baseline: 3.5103x; 3.5103x over previous
import functools

import jax
import jax.numpy as jnp
import numpy as np
from jax import lax
from jax.experimental import pallas as pl
from jax.experimental.pallas import tpu as pltpu
from jax.experimental.pallas import tpu_sc as plsc

ROI_LO_I, ROI_HI_I = 40, 400
M_DIPS = 6
MIN_AREA = 1e-05
W_AREA = 1.0
W_CENTROID = 1.0
W_DEPTH = 0.2
UNDERFILL_FACTOR = 2.0
B, L = 64, 2048
HALF = 10
WN = 2 * HALF + 1
NEG = float("-inf")

NSUB = 16
ROWS_PER = B // NSUB


def _log_taps():
    sigma = 2.0
    radius = int(max(1.0, 3.0 * sigma))
    x = np.arange(-radius, radius + 1, dtype=np.float32)
    s2 = np.float32(sigma * sigma)
    g = np.exp(-(x ** 2) / (2.0 * s2)).astype(np.float32)
    taps = ((x ** 2 - s2) / s2 ** 2 * g).astype(np.float32)
    taps = (taps - taps.mean()).astype(np.float32)
    return taps


_TAPS = _log_taps()
_PAD = 128
_LP = L + 2 * _PAD


def _detect_body(t_ref, centers_ref, valid_ref, pad_ref, pool_ref):
    t = t_ref[:]
    pad_ref[:] = jnp.zeros((B, _LP), jnp.float32)
    pad_ref[:, _PAD:_PAD + L] = t
    acc = jnp.zeros((B, L), jnp.float32)
    for k in range(_TAPS.shape[0]):
        acc = acc + float(_TAPS[k]) * pad_ref[:, _PAD - 6 + k:_PAD - 6 + k + L]
    col = lax.broadcasted_iota(jnp.int32, (B, L), 1)
    roi = ((col >= ROI_LO_I) & (col <= ROI_HI_I)).astype(jnp.float32)
    scores = -acc * roi

    pool_ref[:] = jnp.full((B, _LP), NEG, jnp.float32)
    pool_ref[:, _PAD:_PAD + L] = scores
    pooled = pool_ref[:, _PAD - 5:_PAD - 5 + L]
    for d in range(1, 11):
        pooled = jnp.maximum(pooled, pool_ref[:, _PAD - 5 + d:_PAD - 5 + d + L])

    mean = jnp.mean(scores, axis=1, keepdims=True)
    keep = (scores == pooled) & (scores > mean)
    masked = jnp.where(keep, scores, NEG)

    col16 = lax.broadcasted_iota(jnp.int32, (B, 16), 1)
    centers16 = jnp.zeros((B, 16), jnp.int32)
    valid16 = jnp.zeros((B, 16), jnp.float32)
    for tk in range(M_DIPS):
        m = jnp.max(masked, axis=1, keepdims=True)
        ismax = masked == m
        c = jnp.min(jnp.where(ismax, col, L), axis=1, keepdims=True)
        centers16 = jnp.where(col16 == tk, c, centers16)
        v = (m > NEG).astype(jnp.float32)
        valid16 = jnp.where(col16 == tk, v, valid16)
        masked = jnp.where(col == c, NEG, masked)
    centers_ref[:] = centers16
    valid_ref[:] = valid16


def _detect(target):
    return pl.pallas_call(
        _detect_body,
        out_shape=[
            jax.ShapeDtypeStruct((B, 16), jnp.int32),
            jax.ShapeDtypeStruct((B, 16), jnp.float32),
        ],
        scratch_shapes=[
            pltpu.VMEM((B, _LP), jnp.float32),
            pltpu.VMEM((B, _LP), jnp.float32),
        ],
    )(target)


def _sc_gather_body(tgt_hbm, pred_hbm, centers_hbm, tv_hbm, pv_hbm,
                    trow, prow, crow, twin, pwin):
    wid = lax.axis_index("s")
    for rlocal in range(ROWS_PER):
        row = wid * ROWS_PER + rlocal
        pltpu.sync_copy(tgt_hbm.at[row], trow)
        pltpu.sync_copy(pred_hbm.at[row], prow)
        pltpu.sync_copy(centers_hbm.at[row], crow)
        c = crow[:]
        s = jnp.maximum(c - HALF, 0)
        e = jnp.minimum(c + HALF, L - 1)
        for j in range(WN):
            idx = jnp.minimum(s + j, e)
            twin[j] = plsc.load_gather(trow, [idx])
            pwin[j] = plsc.load_gather(prow, [idx])
        pltpu.sync_copy(twin, tv_hbm.at[:, row])
        pltpu.sync_copy(pwin, pv_hbm.at[:, row])


@functools.cache
def _sc_gather():
  return pl.kernel(
    _sc_gather_body,
    out_type=[
        jax.ShapeDtypeStruct((WN, B, 16), jnp.float32),
        jax.ShapeDtypeStruct((WN, B, 16), jnp.float32),
    ],
    mesh=plsc.VectorSubcoreMesh(core_axis_name="c", subcore_axis_name="s",
                                num_cores=1, num_subcores=NSUB),
    compiler_params=pltpu.CompilerParams(needs_layout_passes=False),
    scratch_types=[
        pltpu.VMEM((L,), jnp.float32),
        pltpu.VMEM((L,), jnp.float32),
        pltpu.VMEM((16,), jnp.int32),
        pltpu.VMEM((WN, 16), jnp.float32),
        pltpu.VMEM((WN, 16), jnp.float32),
    ],
  )


def _terms_body(tv_ref, pv_ref, centers_ref, valid_ref, out_ref):
    c = centers_ref[:]
    vld = valid_ref[:]
    s = jnp.maximum(c - HALF, 0)
    e = jnp.minimum(c + HALF, L - 1)
    n = e - s
    nf = n.astype(jnp.float32)
    lam_s = 300.0 + 0.5 * s.astype(jnp.float32)
    lam_e = 300.0 + 0.5 * e.astype(jnp.float32)
    dlam = lam_e - lam_s + 1e-6
    ts = tv_ref[0]
    te = tv_ref[WN - 1]
    ps = pv_ref[0]
    pe = pv_ref[WN - 1]
    zero = jnp.zeros((B, 16), jnp.float32)
    area_t = zero
    area_p = zero
    ct_num = zero
    ct_den = zero
    cp_num = zero
    cp_den = zero
    dsum = zero
    prev_dt = zero
    prev_dp = zero
    prev_lseg = zero
    for j in range(WN):
        idx = jnp.minimum(s + j, e)
        lseg = 300.0 + 0.5 * idx.astype(jnp.float32)
        tt = (lseg - lam_s) / dlam
        cont_t = jnp.maximum((1.0 - tt) * ts + tt * te, 1e-6)
        cont_p = jnp.maximum((1.0 - tt) * ps + tt * pe, 1e-6)
        tv = tv_ref[j]
        pv = pv_ref[j]
        dt = jnp.clip(1.0 - jnp.clip(tv / cont_t, 0.0, 2.0), 0.0, 1.0)
        dp = jnp.clip(1.0 - jnp.clip(pv / cont_p, 0.0, 2.0), 0.0, 1.0)
        jf = float(j)
        pm = jf <= nf
        if j > 0:
            sm = (jf - 1.0) < nf
            dl = lseg - prev_lseg
            area_t = area_t + jnp.where(sm, (dt + prev_dt) * 0.5 * dl, 0.0)
            area_p = area_p + jnp.where(sm, (dp + prev_dp) * 0.5 * dl, 0.0)
        wt = dt + 1e-7
        wp = dp + 1e-7
        ct_num = ct_num + jnp.where(pm, lseg * wt, 0.0)
        ct_den = ct_den + jnp.where(pm, wt, 0.0)
        cp_num = cp_num + jnp.where(pm, lseg * wp, 0.0)
        cp_den = cp_den + jnp.where(pm, wp, 0.0)
        dsum = dsum + jnp.where(pm, jnp.abs(dp - dt), 0.0)
        prev_dt = dt
        prev_dp = dp
        prev_lseg = lseg
    rel_err = jnp.abs(area_p - area_t) / (area_t + 1e-7)
    underfill = jnp.maximum(area_t - area_p, 0.0) / (area_t + 1e-7)
    area_term = rel_err + (UNDERFILL_FACTOR - 1.0) * underfill
    centroid_term = jnp.abs(cp_num / cp_den - ct_num / ct_den)
    depth_term = dsum / (nf + 1.0)
    valid = (vld > 0.5) & (e > s) & jnp.logical_not(area_t < MIN_AREA)
    cnt = jnp.sum(jnp.where(valid, 1.0, 0.0))
    a = jnp.sum(jnp.where(valid, area_term, 0.0))
    cc = jnp.sum(jnp.where(valid, centroid_term, 0.0))
    dd = jnp.sum(jnp.where(valid, depth_term, 0.0))
    den = jnp.maximum(cnt, 1.0)
    num = W_AREA * a + W_CENTROID * cc + W_DEPTH * dd
    loss = jnp.full((1, 1), num) / jnp.full((1, 1), den)
    loss = jnp.where(jnp.full((1, 1), cnt) > 0.0, loss,
                     jnp.zeros((1, 1), jnp.float32))
    out_ref[:] = loss


def _terms(tv, pv, centers, valid):
    return pl.pallas_call(
        _terms_body,
        out_shape=jax.ShapeDtypeStruct((1, 1), jnp.float32),
    )(tv, pv, centers, valid)


def kernel(prediction, target, lam_nm):
    del lam_nm
    pred = prediction.astype(jnp.float32)
    tgt = target.astype(jnp.float32)
    centers, valid = _detect(tgt)
    tv, pv = _sc_gather()(tgt, pred, centers)
    loss = _terms(tv, pv, centers, valid)
    return loss.reshape(())


# speedup vs baseline: 4.1984x; 1.1960x over previous
import functools

import jax
import jax.numpy as jnp
import numpy as np
from jax import lax
from jax.experimental import pallas as pl
from jax.experimental.pallas import tpu as pltpu
from jax.experimental.pallas import tpu_sc as plsc

ROI_LO_I, ROI_HI_I = 40, 400
M_DIPS = 6
MIN_AREA = 1e-05
W_AREA = 1.0
W_CENTROID = 1.0
W_DEPTH = 0.2
UNDERFILL_FACTOR = 2.0
B, L = 64, 2048
HALF = 10
WN = 2 * HALF + 1
NEG = float("-inf")

NSUB = 16
ROWS_PER = B // NSUB


def _log_taps():
    sigma = 2.0
    radius = int(max(1.0, 3.0 * sigma))
    x = np.arange(-radius, radius + 1, dtype=np.float32)
    s2 = np.float32(sigma * sigma)
    g = np.exp(-(x ** 2) / (2.0 * s2)).astype(np.float32)
    taps = ((x ** 2 - s2) / s2 ** 2 * g).astype(np.float32)
    taps = (taps - taps.mean()).astype(np.float32)
    return taps


_TAPS = _log_taps()
_PAD = 128
_LP = L + 2 * _PAD


def _detect_body(t_ref, centers_ref, valid_ref, pad_ref, pool_ref):
    t = t_ref[:]
    pad_ref[:] = jnp.zeros((B, _LP), jnp.float32)
    pad_ref[:, _PAD:_PAD + L] = t
    acc = jnp.zeros((B, L), jnp.float32)
    for k in range(_TAPS.shape[0]):
        acc = acc + float(_TAPS[k]) * pad_ref[:, _PAD - 6 + k:_PAD - 6 + k + L]
    col = lax.broadcasted_iota(jnp.int32, (B, L), 1)
    roi = ((col >= ROI_LO_I) & (col <= ROI_HI_I)).astype(jnp.float32)
    scores = -acc * roi

    pool_ref[:] = jnp.full((B, _LP), NEG, jnp.float32)
    pool_ref[:, _PAD:_PAD + L] = scores
    pooled = pool_ref[:, _PAD - 5:_PAD - 5 + L]
    for d in range(1, 11):
        pooled = jnp.maximum(pooled, pool_ref[:, _PAD - 5 + d:_PAD - 5 + d + L])

    mean = jnp.mean(scores, axis=1, keepdims=True)
    keep = (scores == pooled) & (scores > mean)
    masked = jnp.where(keep, scores, NEG)

    col16 = lax.broadcasted_iota(jnp.int32, (B, 16), 1)
    centers16 = jnp.zeros((B, 16), jnp.int32)
    valid16 = jnp.zeros((B, 16), jnp.float32)
    for tk in range(M_DIPS):
        m = jnp.max(masked, axis=1, keepdims=True)
        ismax = masked == m
        c = jnp.min(jnp.where(ismax, col, L), axis=1, keepdims=True)
        centers16 = jnp.where(col16 == tk, c, centers16)
        v = (m > NEG).astype(jnp.float32)
        valid16 = jnp.where(col16 == tk, v, valid16)
        masked = jnp.where(col == c, NEG, masked)
    centers_ref[:] = centers16
    valid_ref[:] = valid16


def _detect(target):
    return pl.pallas_call(
        _detect_body,
        out_shape=[
            jax.ShapeDtypeStruct((B, 16), jnp.int32),
            jax.ShapeDtypeStruct((B, 16), jnp.float32),
        ],
        scratch_shapes=[
            pltpu.VMEM((B, _LP), jnp.float32),
            pltpu.VMEM((B, _LP), jnp.float32),
        ],
    )(target)


def _sc_gather_body(tgt_hbm, pred_hbm, centers_hbm, tv_hbm, pv_hbm,
                    trows, prows, crows, twin, pwin, sem):
    wid = lax.axis_index("s")
    base = wid * ROWS_PER
    copies = [pltpu.async_copy(centers_hbm.at[pl.ds(base, ROWS_PER)], crows,
                               sem)]
    for r in range(ROWS_PER):
        copies.append(pltpu.async_copy(tgt_hbm.at[pl.ds(base + r, 1)],
                                       trows.at[pl.ds(r, 1)], sem))
        copies.append(pltpu.async_copy(pred_hbm.at[pl.ds(base + r, 1)],
                                       prows.at[pl.ds(r, 1)], sem))
    for cp in copies:
        cp.wait()
    for r in range(ROWS_PER):
        c = crows[r]
        s = jnp.maximum(c - HALF, 0)
        e = jnp.minimum(c + HALF, L - 1)
        rvec = jnp.full((16,), r, jnp.int32)
        for j in range(WN):
            idx = jnp.minimum(s + j, e)
            twin[j, r] = plsc.load_gather(trows, [rvec, idx])
            pwin[j, r] = plsc.load_gather(prows, [rvec, idx])
    o1 = pltpu.async_copy(twin, tv_hbm.at[:, pl.ds(base, ROWS_PER)], sem)
    o2 = pltpu.async_copy(pwin, pv_hbm.at[:, pl.ds(base, ROWS_PER)], sem)
    o1.wait()
    o2.wait()


@functools.cache
def _sc_gather():
  return pl.kernel(
    _sc_gather_body,
    out_type=[
        jax.ShapeDtypeStruct((WN, B, 16), jnp.float32),
        jax.ShapeDtypeStruct((WN, B, 16), jnp.float32),
    ],
    mesh=plsc.VectorSubcoreMesh(core_axis_name="c", subcore_axis_name="s",
                                num_cores=1, num_subcores=NSUB),
    compiler_params=pltpu.CompilerParams(needs_layout_passes=False),
    scratch_types=[
        pltpu.VMEM((ROWS_PER, L), jnp.float32),
        pltpu.VMEM((ROWS_PER, L), jnp.float32),
        pltpu.VMEM((ROWS_PER, 16), jnp.int32),
        pltpu.VMEM((WN, ROWS_PER, 16), jnp.float32),
        pltpu.VMEM((WN, ROWS_PER, 16), jnp.float32),
        pltpu.SemaphoreType.DMA,
    ],
  )


def _terms_body(tv_ref, pv_ref, centers_ref, valid_ref, out_ref):
    c = centers_ref[:]
    vld = valid_ref[:]
    s = jnp.maximum(c - HALF, 0)
    e = jnp.minimum(c + HALF, L - 1)
    n = e - s
    nf = n.astype(jnp.float32)
    lam_s = 300.0 + 0.5 * s.astype(jnp.float32)
    lam_e = 300.0 + 0.5 * e.astype(jnp.float32)
    dlam = lam_e - lam_s + 1e-6
    ts = tv_ref[0]
    te = tv_ref[WN - 1]
    ps = pv_ref[0]
    pe = pv_ref[WN - 1]
    zero = jnp.zeros((B, 16), jnp.float32)
    area_t = zero
    area_p = zero
    ct_num = zero
    ct_den = zero
    cp_num = zero
    cp_den = zero
    dsum = zero
    prev_dt = zero
    prev_dp = zero
    prev_lseg = zero
    for j in range(WN):
        idx = jnp.minimum(s + j, e)
        lseg = 300.0 + 0.5 * idx.astype(jnp.float32)
        tt = (lseg - lam_s) / dlam
        cont_t = jnp.maximum((1.0 - tt) * ts + tt * te, 1e-6)
        cont_p = jnp.maximum((1.0 - tt) * ps + tt * pe, 1e-6)
        tv = tv_ref[j]
        pv = pv_ref[j]
        dt = jnp.clip(1.0 - jnp.clip(tv / cont_t, 0.0, 2.0), 0.0, 1.0)
        dp = jnp.clip(1.0 - jnp.clip(pv / cont_p, 0.0, 2.0), 0.0, 1.0)
        jf = float(j)
        pm = jf <= nf
        if j > 0:
            sm = (jf - 1.0) < nf
            dl = lseg - prev_lseg
            area_t = area_t + jnp.where(sm, (dt + prev_dt) * 0.5 * dl, 0.0)
            area_p = area_p + jnp.where(sm, (dp + prev_dp) * 0.5 * dl, 0.0)
        wt = dt + 1e-7
        wp = dp + 1e-7
        ct_num = ct_num + jnp.where(pm, lseg * wt, 0.0)
        ct_den = ct_den + jnp.where(pm, wt, 0.0)
        cp_num = cp_num + jnp.where(pm, lseg * wp, 0.0)
        cp_den = cp_den + jnp.where(pm, wp, 0.0)
        dsum = dsum + jnp.where(pm, jnp.abs(dp - dt), 0.0)
        prev_dt = dt
        prev_dp = dp
        prev_lseg = lseg
    rel_err = jnp.abs(area_p - area_t) / (area_t + 1e-7)
    underfill = jnp.maximum(area_t - area_p, 0.0) / (area_t + 1e-7)
    area_term = rel_err + (UNDERFILL_FACTOR - 1.0) * underfill
    centroid_term = jnp.abs(cp_num / cp_den - ct_num / ct_den)
    depth_term = dsum / (nf + 1.0)
    valid = (vld > 0.5) & (e > s) & jnp.logical_not(area_t < MIN_AREA)
    cnt = jnp.sum(jnp.where(valid, 1.0, 0.0))
    a = jnp.sum(jnp.where(valid, area_term, 0.0))
    cc = jnp.sum(jnp.where(valid, centroid_term, 0.0))
    dd = jnp.sum(jnp.where(valid, depth_term, 0.0))
    den = jnp.maximum(cnt, 1.0)
    num = W_AREA * a + W_CENTROID * cc + W_DEPTH * dd
    loss = jnp.full((1, 1), num) / jnp.full((1, 1), den)
    loss = jnp.where(jnp.full((1, 1), cnt) > 0.0, loss,
                     jnp.zeros((1, 1), jnp.float32))
    out_ref[:] = loss


def _terms(tv, pv, centers, valid):
    return pl.pallas_call(
        _terms_body,
        out_shape=jax.ShapeDtypeStruct((1, 1), jnp.float32),
    )(tv, pv, centers, valid)


def kernel(prediction, target, lam_nm):
    del lam_nm
    pred = prediction.astype(jnp.float32)
    tgt = target.astype(jnp.float32)
    centers, valid = _detect(tgt)
    tv, pv = _sc_gather()(tgt, pred, centers)
    loss = _terms(tv, pv, centers, valid)
    return loss.reshape(())


# speedup vs baseline: 4.4315x; 1.0555x over previous
import functools

import jax
import jax.numpy as jnp
import numpy as np
from jax import lax
from jax.experimental import pallas as pl
from jax.experimental.pallas import tpu as pltpu
from jax.experimental.pallas import tpu_sc as plsc

ROI_LO_I, ROI_HI_I = 40, 400
M_DIPS = 6
MIN_AREA = 1e-05
W_AREA = 1.0
W_CENTROID = 1.0
W_DEPTH = 0.2
UNDERFILL_FACTOR = 2.0
B, L = 64, 2048
HALF = 10
WN = 2 * HALF + 1
NEG = float("-inf")

NSUB = 16
ROWS_PER = B // NSUB


def _log_taps():
    sigma = 2.0
    radius = int(max(1.0, 3.0 * sigma))
    x = np.arange(-radius, radius + 1, dtype=np.float32)
    s2 = np.float32(sigma * sigma)
    g = np.exp(-(x ** 2) / (2.0 * s2)).astype(np.float32)
    taps = ((x ** 2 - s2) / s2 ** 2 * g).astype(np.float32)
    taps = (taps - taps.mean()).astype(np.float32)
    return taps


_TAPS = _log_taps()
_PAD = 128
_LP = L + 2 * _PAD


BK0 = 32
BW = 512


def _detect_body(t_ref, centers_ref, valid_ref, pool_ref):
    acc = jnp.zeros((B, BW), jnp.float32)
    for k in range(_TAPS.shape[0]):
        off = BK0 - 6 + k
        acc = acc + float(_TAPS[k]) * t_ref[:, off:off + BW]
    colb = lax.broadcasted_iota(jnp.int32, (B, BW), 1)
    roi = ((colb >= ROI_LO_I - BK0) & (colb <= ROI_HI_I - BK0)).astype(
        jnp.float32)
    scores = -acc * roi

    pool_ref[:] = jnp.zeros((B, BW + 128), jnp.float32)
    pool_ref[:, 64:64 + BW] = scores
    pooled = pool_ref[:, 59:59 + BW]
    for d in range(1, 11):
        pooled = jnp.maximum(pooled, pool_ref[:, 59 + d:59 + d + BW])

    mean = jnp.sum(scores, axis=1, keepdims=True) * (1.0 / L)
    keep = (scores == pooled) & (scores > mean)
    masked = jnp.where(keep & (scores > 0.0), scores, NEG)

    cbs = []
    pvs = []
    for tk in range(M_DIPS):
        m = jnp.max(masked, axis=1, keepdims=True)
        ismax = masked == m
        cb = jnp.min(jnp.where(ismax, colb, BW), axis=1, keepdims=True)
        cbs.append(cb)
        pvs.append(m > NEG)
        masked = jnp.where(colb == cb, NEG, masked)

    p_cnt = jnp.zeros((B, 1), jnp.int32)
    for pv in pvs:
        p_cnt = p_cnt + pv.astype(jnp.int32)
    neg_mean = mean < 0.0

    col16 = lax.broadcasted_iota(jnp.int32, (B, 16), 1)
    centers16 = jnp.zeros((B, 16), jnp.int32)
    valid16 = jnp.zeros((B, 16), jnp.float32)
    for tk in range(M_DIPS):
        c = jnp.where(pvs[tk], BK0 + cbs[tk], tk - p_cnt)
        v = (pvs[tk] | neg_mean).astype(jnp.float32)
        centers16 = jnp.where(col16 == tk, c, centers16)
        valid16 = jnp.where(col16 == tk, v, valid16)
    centers_ref[:] = centers16
    valid_ref[:] = valid16


def _detect(target):
    return pl.pallas_call(
        _detect_body,
        out_shape=[
            jax.ShapeDtypeStruct((B, 16), jnp.int32),
            jax.ShapeDtypeStruct((B, 16), jnp.float32),
        ],
        scratch_shapes=[
            pltpu.VMEM((B, BW + 128), jnp.float32),
        ],
    )(target)


def _sc_gather_body(tgt_hbm, pred_hbm, centers_hbm, tv_hbm, pv_hbm,
                    trows, prows, crows, twin, pwin, sem):
    wid = lax.axis_index("s")
    base = wid * ROWS_PER
    copies = [pltpu.async_copy(centers_hbm.at[pl.ds(base, ROWS_PER)], crows,
                               sem)]
    for r in range(ROWS_PER):
        copies.append(pltpu.async_copy(tgt_hbm.at[pl.ds(base + r, 1)],
                                       trows.at[pl.ds(r, 1)], sem))
        copies.append(pltpu.async_copy(pred_hbm.at[pl.ds(base + r, 1)],
                                       prows.at[pl.ds(r, 1)], sem))
    for cp in copies:
        cp.wait()
    for r in range(ROWS_PER):
        c = crows[r]
        s = jnp.maximum(c - HALF, 0)
        e = jnp.minimum(c + HALF, L - 1)
        rvec = jnp.full((16,), r, jnp.int32)
        for j in range(WN):
            idx = jnp.minimum(s + j, e)
            twin[j, r] = plsc.load_gather(trows, [rvec, idx])
            pwin[j, r] = plsc.load_gather(prows, [rvec, idx])
    o1 = pltpu.async_copy(twin, tv_hbm.at[:, pl.ds(base, ROWS_PER)], sem)
    o2 = pltpu.async_copy(pwin, pv_hbm.at[:, pl.ds(base, ROWS_PER)], sem)
    o1.wait()
    o2.wait()


@functools.cache
def _sc_gather():
  return pl.kernel(
    _sc_gather_body,
    out_type=[
        jax.ShapeDtypeStruct((WN, B, 16), jnp.float32),
        jax.ShapeDtypeStruct((WN, B, 16), jnp.float32),
    ],
    mesh=plsc.VectorSubcoreMesh(core_axis_name="c", subcore_axis_name="s",
                                num_cores=1, num_subcores=NSUB),
    compiler_params=pltpu.CompilerParams(needs_layout_passes=False),
    scratch_types=[
        pltpu.VMEM((ROWS_PER, L), jnp.float32),
        pltpu.VMEM((ROWS_PER, L), jnp.float32),
        pltpu.VMEM((ROWS_PER, 16), jnp.int32),
        pltpu.VMEM((WN, ROWS_PER, 16), jnp.float32),
        pltpu.VMEM((WN, ROWS_PER, 16), jnp.float32),
        pltpu.SemaphoreType.DMA,
    ],
  )


def _terms_body(tv_ref, pv_ref, centers_ref, valid_ref, out_ref):
    c = centers_ref[:]
    vld = valid_ref[:]
    s = jnp.maximum(c - HALF, 0)
    e = jnp.minimum(c + HALF, L - 1)
    n = e - s
    nf = n.astype(jnp.float32)
    lam_s = 300.0 + 0.5 * s.astype(jnp.float32)
    lam_e = 300.0 + 0.5 * e.astype(jnp.float32)
    dlam = lam_e - lam_s + 1e-6
    ts = tv_ref[0]
    te = tv_ref[WN - 1]
    ps = pv_ref[0]
    pe = pv_ref[WN - 1]
    zero = jnp.zeros((8, 128), jnp.float32)
    area_t = zero
    area_p = zero
    ct_num = zero
    ct_den = zero
    cp_num = zero
    cp_den = zero
    dsum = zero
    prev_dt = zero
    prev_dp = zero
    prev_lseg = zero
    for j in range(WN):
        idx = jnp.minimum(s + j, e)
        lseg = 300.0 + 0.5 * idx.astype(jnp.float32)
        tt = (lseg - lam_s) / dlam
        cont_t = jnp.maximum((1.0 - tt) * ts + tt * te, 1e-6)
        cont_p = jnp.maximum((1.0 - tt) * ps + tt * pe, 1e-6)
        tv = tv_ref[j]
        pv = pv_ref[j]
        dt = jnp.clip(1.0 - jnp.clip(tv / cont_t, 0.0, 2.0), 0.0, 1.0)
        dp = jnp.clip(1.0 - jnp.clip(pv / cont_p, 0.0, 2.0), 0.0, 1.0)
        jf = float(j)
        pm = jf <= nf
        if j > 0:
            sm = (jf - 1.0) < nf
            dl = lseg - prev_lseg
            area_t = area_t + jnp.where(sm, (dt + prev_dt) * 0.5 * dl, 0.0)
            area_p = area_p + jnp.where(sm, (dp + prev_dp) * 0.5 * dl, 0.0)
        wt = dt + 1e-7
        wp = dp + 1e-7
        ct_num = ct_num + jnp.where(pm, lseg * wt, 0.0)
        ct_den = ct_den + jnp.where(pm, wt, 0.0)
        cp_num = cp_num + jnp.where(pm, lseg * wp, 0.0)
        cp_den = cp_den + jnp.where(pm, wp, 0.0)
        dsum = dsum + jnp.where(pm, jnp.abs(dp - dt), 0.0)
        prev_dt = dt
        prev_dp = dp
        prev_lseg = lseg
    rel_err = jnp.abs(area_p - area_t) / (area_t + 1e-7)
    underfill = jnp.maximum(area_t - area_p, 0.0) / (area_t + 1e-7)
    area_term = rel_err + (UNDERFILL_FACTOR - 1.0) * underfill
    centroid_term = jnp.abs(cp_num / cp_den - ct_num / ct_den)
    depth_term = dsum / (nf + 1.0)
    valid = (vld > 0.5) & (e > s) & jnp.logical_not(area_t < MIN_AREA)
    cnt = jnp.sum(jnp.where(valid, 1.0, 0.0))
    a = jnp.sum(jnp.where(valid, area_term, 0.0))
    cc = jnp.sum(jnp.where(valid, centroid_term, 0.0))
    dd = jnp.sum(jnp.where(valid, depth_term, 0.0))
    den = jnp.maximum(cnt, 1.0)
    num = W_AREA * a + W_CENTROID * cc + W_DEPTH * dd
    loss = jnp.full((1, 1), num) / jnp.full((1, 1), den)
    loss = jnp.where(jnp.full((1, 1), cnt) > 0.0, loss,
                     jnp.zeros((1, 1), jnp.float32))
    out_ref[:] = loss


def _terms(tv, pv, centers, valid):
    return pl.pallas_call(
        _terms_body,
        out_shape=jax.ShapeDtypeStruct((1, 1), jnp.float32),
    )(tv, pv, centers, valid)


def kernel(prediction, target, lam_nm):
    del lam_nm
    pred = prediction.astype(jnp.float32)
    tgt = target.astype(jnp.float32)
    centers, valid = _detect(tgt)
    tv, pv = _sc_gather()(tgt, pred, centers)
    loss = _terms(tv.reshape(WN, 8, 128), pv.reshape(WN, 8, 128),
                  centers.reshape(8, 128), valid.reshape(8, 128))
    return loss.reshape(())


# speedup vs baseline: 4.4944x; 1.0142x over previous
import functools

import jax
import jax.numpy as jnp
import numpy as np
from jax import lax
from jax.experimental import pallas as pl
from jax.experimental.pallas import tpu as pltpu
from jax.experimental.pallas import tpu_sc as plsc

ROI_LO_I, ROI_HI_I = 40, 400
M_DIPS = 6
MIN_AREA = 1e-05
W_AREA = 1.0
W_CENTROID = 1.0
W_DEPTH = 0.2
UNDERFILL_FACTOR = 2.0
B, L = 64, 2048
HALF = 10
WN = 2 * HALF + 1
NEG = float("-inf")

NSUB = 16
NCORE = 2
NW = NCORE * NSUB
ROWS_PER = B // NW


def _log_taps():
    sigma = 2.0
    radius = int(max(1.0, 3.0 * sigma))
    x = np.arange(-radius, radius + 1, dtype=np.float32)
    s2 = np.float32(sigma * sigma)
    g = np.exp(-(x ** 2) / (2.0 * s2)).astype(np.float32)
    taps = ((x ** 2 - s2) / s2 ** 2 * g).astype(np.float32)
    taps = (taps - taps.mean()).astype(np.float32)
    return taps


_TAPS = _log_taps()
_PAD = 128
_LP = L + 2 * _PAD


BK0 = 32
BW = 512


def _detect_body(t_ref, centers_ref, valid_ref, pool_ref):
    acc = jnp.zeros((B, BW), jnp.float32)
    for k in range(_TAPS.shape[0]):
        off = BK0 - 6 + k
        acc = acc + float(_TAPS[k]) * t_ref[:, off:off + BW]
    colb = lax.broadcasted_iota(jnp.int32, (B, BW), 1)
    roi = ((colb >= ROI_LO_I - BK0) & (colb <= ROI_HI_I - BK0)).astype(
        jnp.float32)
    scores = -acc * roi

    pool_ref[:] = jnp.zeros((B, BW + 128), jnp.float32)
    pool_ref[:, 64:64 + BW] = scores
    pooled = pool_ref[:, 59:59 + BW]
    for d in range(1, 11):
        pooled = jnp.maximum(pooled, pool_ref[:, 59 + d:59 + d + BW])

    mean = jnp.sum(scores, axis=1, keepdims=True) * (1.0 / L)
    keep = (scores == pooled) & (scores > mean)
    masked = jnp.where(keep & (scores > 0.0), scores, NEG)

    cbs = []
    pvs = []
    for tk in range(M_DIPS):
        m = jnp.max(masked, axis=1, keepdims=True)
        ismax = masked == m
        cb = jnp.min(jnp.where(ismax, colb, BW), axis=1, keepdims=True)
        cbs.append(cb)
        pvs.append(m > NEG)
        masked = jnp.where(colb == cb, NEG, masked)

    p_cnt = jnp.zeros((B, 1), jnp.int32)
    for pv in pvs:
        p_cnt = p_cnt + pv.astype(jnp.int32)
    neg_mean = mean < 0.0

    col16 = lax.broadcasted_iota(jnp.int32, (B, 16), 1)
    centers16 = jnp.zeros((B, 16), jnp.int32)
    valid16 = jnp.zeros((B, 16), jnp.float32)
    for tk in range(M_DIPS):
        c = jnp.where(pvs[tk], BK0 + cbs[tk], tk - p_cnt)
        v = (pvs[tk] | neg_mean).astype(jnp.float32)
        centers16 = jnp.where(col16 == tk, c, centers16)
        valid16 = jnp.where(col16 == tk, v, valid16)
    centers_ref[:] = centers16
    valid_ref[:] = valid16


def _detect(target):
    return pl.pallas_call(
        _detect_body,
        out_shape=[
            jax.ShapeDtypeStruct((B, 16), jnp.int32),
            jax.ShapeDtypeStruct((B, 16), jnp.float32),
        ],
        scratch_shapes=[
            pltpu.VMEM((B, BW + 128), jnp.float32),
        ],
    )(target)


def _sc_gather_body(tgt_hbm, pred_hbm, centers_hbm, tv_hbm, pv_hbm,
                    trows, prows, crows, twin, pwin, sem):
    wid = lax.axis_index("s") * NCORE + lax.axis_index("c")
    base = wid * ROWS_PER
    copies = [
        pltpu.async_copy(centers_hbm.at[pl.ds(base, ROWS_PER)], crows, sem),
        pltpu.async_copy(tgt_hbm.at[pl.ds(base, ROWS_PER)], trows, sem),
        pltpu.async_copy(pred_hbm.at[pl.ds(base, ROWS_PER)], prows, sem),
    ]
    for cp in copies:
        cp.wait()
    for r in range(ROWS_PER):
        c = crows[r]
        s = jnp.maximum(c - HALF, 0)
        e = jnp.minimum(c + HALF, L - 1)
        rvec = jnp.full((16,), r, jnp.int32)
        for j in range(WN):
            idx = jnp.minimum(s + j, e)
            twin[j, r] = plsc.load_gather(trows, [rvec, idx])
            pwin[j, r] = plsc.load_gather(prows, [rvec, idx])
    o1 = pltpu.async_copy(twin, tv_hbm.at[:, pl.ds(base, ROWS_PER)], sem)
    o2 = pltpu.async_copy(pwin, pv_hbm.at[:, pl.ds(base, ROWS_PER)], sem)
    o1.wait()
    o2.wait()


@functools.cache
def _sc_gather():
  return pl.kernel(
    _sc_gather_body,
    out_type=[
        jax.ShapeDtypeStruct((WN, B, 16), jnp.float32),
        jax.ShapeDtypeStruct((WN, B, 16), jnp.float32),
    ],
    mesh=plsc.VectorSubcoreMesh(core_axis_name="c", subcore_axis_name="s",
                                num_cores=NCORE, num_subcores=NSUB),
    compiler_params=pltpu.CompilerParams(needs_layout_passes=False),
    scratch_types=[
        pltpu.VMEM((ROWS_PER, L), jnp.float32),
        pltpu.VMEM((ROWS_PER, L), jnp.float32),
        pltpu.VMEM((ROWS_PER, 16), jnp.int32),
        pltpu.VMEM((WN, ROWS_PER, 16), jnp.float32),
        pltpu.VMEM((WN, ROWS_PER, 16), jnp.float32),
        pltpu.SemaphoreType.DMA,
    ],
  )


def _terms_body(tv_ref, pv_ref, centers_ref, valid_ref, out_ref):
    c = centers_ref[:]
    vld = valid_ref[:]
    s = jnp.maximum(c - HALF, 0)
    e = jnp.minimum(c + HALF, L - 1)
    n = e - s
    nf = n.astype(jnp.float32)
    lam_s = 300.0 + 0.5 * s.astype(jnp.float32)
    lam_e = 300.0 + 0.5 * e.astype(jnp.float32)
    dlam = lam_e - lam_s + 1e-6
    ts = tv_ref[0]
    te = tv_ref[WN - 1]
    ps = pv_ref[0]
    pe = pv_ref[WN - 1]
    zero = jnp.zeros((8, 128), jnp.float32)
    area_t = zero
    area_p = zero
    ct_num = zero
    ct_den = zero
    cp_num = zero
    cp_den = zero
    dsum = zero
    prev_dt = zero
    prev_dp = zero
    prev_lseg = zero
    for j in range(WN):
        idx = jnp.minimum(s + j, e)
        lseg = 300.0 + 0.5 * idx.astype(jnp.float32)
        tt = (lseg - lam_s) / dlam
        cont_t = jnp.maximum((1.0 - tt) * ts + tt * te, 1e-6)
        cont_p = jnp.maximum((1.0 - tt) * ps + tt * pe, 1e-6)
        tv = tv_ref[j]
        pv = pv_ref[j]
        dt = jnp.clip(1.0 - jnp.clip(tv / cont_t, 0.0, 2.0), 0.0, 1.0)
        dp = jnp.clip(1.0 - jnp.clip(pv / cont_p, 0.0, 2.0), 0.0, 1.0)
        jf = float(j)
        pm = jf <= nf
        if j > 0:
            sm = (jf - 1.0) < nf
            dl = lseg - prev_lseg
            area_t = area_t + jnp.where(sm, (dt + prev_dt) * 0.5 * dl, 0.0)
            area_p = area_p + jnp.where(sm, (dp + prev_dp) * 0.5 * dl, 0.0)
        wt = dt + 1e-7
        wp = dp + 1e-7
        ct_num = ct_num + jnp.where(pm, lseg * wt, 0.0)
        ct_den = ct_den + jnp.where(pm, wt, 0.0)
        cp_num = cp_num + jnp.where(pm, lseg * wp, 0.0)
        cp_den = cp_den + jnp.where(pm, wp, 0.0)
        dsum = dsum + jnp.where(pm, jnp.abs(dp - dt), 0.0)
        prev_dt = dt
        prev_dp = dp
        prev_lseg = lseg
    rel_err = jnp.abs(area_p - area_t) / (area_t + 1e-7)
    underfill = jnp.maximum(area_t - area_p, 0.0) / (area_t + 1e-7)
    area_term = rel_err + (UNDERFILL_FACTOR - 1.0) * underfill
    centroid_term = jnp.abs(cp_num / cp_den - ct_num / ct_den)
    depth_term = dsum / (nf + 1.0)
    valid = (vld > 0.5) & (e > s) & jnp.logical_not(area_t < MIN_AREA)
    cnt = jnp.sum(jnp.where(valid, 1.0, 0.0))
    a = jnp.sum(jnp.where(valid, area_term, 0.0))
    cc = jnp.sum(jnp.where(valid, centroid_term, 0.0))
    dd = jnp.sum(jnp.where(valid, depth_term, 0.0))
    den = jnp.maximum(cnt, 1.0)
    num = W_AREA * a + W_CENTROID * cc + W_DEPTH * dd
    loss = jnp.full((1, 1), num) / jnp.full((1, 1), den)
    loss = jnp.where(jnp.full((1, 1), cnt) > 0.0, loss,
                     jnp.zeros((1, 1), jnp.float32))
    out_ref[:] = loss


def _terms(tv, pv, centers, valid):
    return pl.pallas_call(
        _terms_body,
        out_shape=jax.ShapeDtypeStruct((1, 1), jnp.float32),
    )(tv, pv, centers, valid)


def kernel(prediction, target, lam_nm):
    del lam_nm
    pred = prediction.astype(jnp.float32)
    tgt = target.astype(jnp.float32)
    centers, valid = _detect(tgt)
    tv, pv = _sc_gather()(tgt, pred, centers)
    loss = _terms(tv.reshape(WN, 8, 128), pv.reshape(WN, 8, 128),
                  centers.reshape(8, 128), valid.reshape(8, 128))
    return loss.reshape(())


# speedup vs baseline: 4.6953x; 1.0447x over previous
import functools

import jax
import jax.numpy as jnp
import numpy as np
from jax import lax
from jax.experimental import pallas as pl
from jax.experimental.pallas import tpu as pltpu
from jax.experimental.pallas import tpu_sc as plsc

ROI_LO_I, ROI_HI_I = 40, 400
M_DIPS = 6
MIN_AREA = 1e-05
W_AREA = 1.0
W_CENTROID = 1.0
W_DEPTH = 0.2
UNDERFILL_FACTOR = 2.0
B, L = 64, 2048
HALF = 10
WN = 2 * HALF + 1
NEG = float("-inf")

NSUB = 16
NCORE = 2
NW = NCORE * NSUB
ROWS_PER = B // NW


def _log_taps():
    sigma = 2.0
    radius = int(max(1.0, 3.0 * sigma))
    x = np.arange(-radius, radius + 1, dtype=np.float32)
    s2 = np.float32(sigma * sigma)
    g = np.exp(-(x ** 2) / (2.0 * s2)).astype(np.float32)
    taps = ((x ** 2 - s2) / s2 ** 2 * g).astype(np.float32)
    taps = (taps - taps.mean()).astype(np.float32)
    return taps


_TAPS = _log_taps()

BK0 = 32
BW = 512


def _detect_body(t_ref, centers_ref, pool_ref):
    acc = jnp.zeros((B, BW), jnp.float32)
    for k in range(_TAPS.shape[0]):
        off = BK0 - 6 + k
        acc = acc + float(_TAPS[k]) * t_ref[:, off:off + BW]
    colb = lax.broadcasted_iota(jnp.int32, (B, BW), 1)
    roi = ((colb >= ROI_LO_I - BK0) & (colb <= ROI_HI_I - BK0)).astype(
        jnp.float32)
    scores = -acc * roi

    pool_ref[:] = jnp.zeros((B, BW + 128), jnp.float32)
    pool_ref[:, 64:64 + BW] = scores
    pooled = pool_ref[:, 59:59 + BW]
    for d in range(1, 11):
        pooled = jnp.maximum(pooled, pool_ref[:, 59 + d:59 + d + BW])

    mean = jnp.sum(scores, axis=1, keepdims=True) * (1.0 / L)
    keep = (scores == pooled) & (scores > mean)
    masked = jnp.where(keep & (scores > 0.0), scores, NEG)

    cbs = []
    pvs = []
    for tk in range(M_DIPS):
        m = jnp.max(masked, axis=1, keepdims=True)
        ismax = masked == m
        cb = jnp.min(jnp.where(ismax, colb, BW), axis=1, keepdims=True)
        cbs.append(cb)
        pvs.append(m > NEG)
        masked = jnp.where(colb == cb, NEG, masked)

    p_cnt = jnp.zeros((B, 1), jnp.int32)
    for pv in pvs:
        p_cnt = p_cnt + pv.astype(jnp.int32)
    neg_mean = mean < 0.0

    col16 = lax.broadcasted_iota(jnp.int32, (B, 16), 1)
    centers16 = jnp.full((B, 16), -1, jnp.int32)
    for tk in range(M_DIPS):
        c = jnp.where(pvs[tk], BK0 + cbs[tk], tk - p_cnt)
        enc = jnp.where(pvs[tk] | neg_mean, c, -1)
        centers16 = jnp.where(col16 == tk, enc, centers16)
    centers_ref[:] = centers16


def _detect(target):
    return pl.pallas_call(
        _detect_body,
        out_shape=jax.ShapeDtypeStruct((B, 16), jnp.int32),
        scratch_shapes=[
            pltpu.VMEM((B, BW + 128), jnp.float32),
        ],
    )(target)


def _sc_gather_body(tgt_hbm, pred_hbm, cenc_hbm, tv_hbm, pv_hbm,
                    trows, prows, crows, twin, pwin, sem):
    wid = lax.axis_index("s") * NCORE + lax.axis_index("c")
    base = wid * ROWS_PER
    copies = [
        pltpu.async_copy(cenc_hbm.at[pl.ds(base, ROWS_PER)], crows, sem),
        pltpu.async_copy(tgt_hbm.at[pl.ds(base, ROWS_PER)], trows, sem),
        pltpu.async_copy(pred_hbm.at[pl.ds(base, ROWS_PER)], prows, sem),
    ]
    for cp in copies:
        cp.wait()
    for r in range(ROWS_PER):
        ce = crows[r]
        c = jnp.maximum(ce, 0)
        s = jnp.maximum(c - HALF, 0)
        e = jnp.minimum(c + HALF, L - 1)
        rvec = jnp.full((16,), r, jnp.int32)
        for j in range(WN):
            idx = jnp.minimum(s + j, e)
            twin[j, r] = plsc.load_gather(trows, [rvec, idx])
            pwin[j, r] = plsc.load_gather(prows, [rvec, idx])
    outs = [
        pltpu.async_copy(twin, tv_hbm.at[:, pl.ds(base, ROWS_PER)], sem),
        pltpu.async_copy(pwin, pv_hbm.at[:, pl.ds(base, ROWS_PER)], sem),
    ]
    for cp in outs:
        cp.wait()


@functools.cache
def _sc_gather():
  return pl.kernel(
    _sc_gather_body,
    out_type=[
        jax.ShapeDtypeStruct((WN, B, 16), jnp.float32),
        jax.ShapeDtypeStruct((WN, B, 16), jnp.float32),
    ],
    mesh=plsc.VectorSubcoreMesh(core_axis_name="c", subcore_axis_name="s",
                                num_cores=NCORE, num_subcores=NSUB),
    compiler_params=pltpu.CompilerParams(needs_layout_passes=False),
    scratch_types=[
        pltpu.VMEM((ROWS_PER, L), jnp.float32),
        pltpu.VMEM((ROWS_PER, L), jnp.float32),
        pltpu.VMEM((ROWS_PER, 16), jnp.int32),
        pltpu.VMEM((WN, ROWS_PER, 16), jnp.float32),
        pltpu.VMEM((WN, ROWS_PER, 16), jnp.float32),
        pltpu.SemaphoreType.DMA,
    ],
  )


def _terms_body(tv_ref, pv_ref, c8_ref, out_ref):
    ce = c8_ref[:]
    vld_b = ce >= 0
    c = jnp.where(vld_b, ce, 0)
    s = jnp.maximum(c - HALF, 0)
    e = jnp.minimum(c + HALF, L - 1)
    n = e - s
    nf = n.astype(jnp.float32)
    lam_s = 300.0 + 0.5 * s.astype(jnp.float32)
    lam_e = 300.0 + 0.5 * e.astype(jnp.float32)
    dlam = lam_e - lam_s + 1e-6
    ts = tv_ref[0]
    te = tv_ref[WN - 1]
    ps = pv_ref[0]
    pe = pv_ref[WN - 1]
    zero = jnp.zeros((B, 16), jnp.float32)
    area_t = zero
    area_p = zero
    ct_num = zero
    ct_den = zero
    cp_num = zero
    cp_den = zero
    dsum = zero
    prev_dt = zero
    prev_dp = zero
    prev_lseg = zero
    for j in range(WN):
        idx = jnp.minimum(s + j, e)
        lseg = 300.0 + 0.5 * idx.astype(jnp.float32)
        tt = (lseg - lam_s) / dlam
        cont_t = jnp.maximum((1.0 - tt) * ts + tt * te, 1e-6)
        cont_p = jnp.maximum((1.0 - tt) * ps + tt * pe, 1e-6)
        tv = tv_ref[j]
        pv = pv_ref[j]
        dt = jnp.clip(1.0 - jnp.clip(tv / cont_t, 0.0, 2.0), 0.0, 1.0)
        dp = jnp.clip(1.0 - jnp.clip(pv / cont_p, 0.0, 2.0), 0.0, 1.0)
        jf = float(j)
        pm = jf <= nf
        if j > 0:
            sm = (jf - 1.0) < nf
            dl = lseg - prev_lseg
            area_t = area_t + jnp.where(sm, (dt + prev_dt) * 0.5 * dl, 0.0)
            area_p = area_p + jnp.where(sm, (dp + prev_dp) * 0.5 * dl, 0.0)
        wt = dt + 1e-7
        wp = dp + 1e-7
        ct_num = ct_num + jnp.where(pm, lseg * wt, 0.0)
        ct_den = ct_den + jnp.where(pm, wt, 0.0)
        cp_num = cp_num + jnp.where(pm, lseg * wp, 0.0)
        cp_den = cp_den + jnp.where(pm, wp, 0.0)
        dsum = dsum + jnp.where(pm, jnp.abs(dp - dt), 0.0)
        prev_dt = dt
        prev_dp = dp
        prev_lseg = lseg
    rel_err = jnp.abs(area_p - area_t) / (area_t + 1e-7)
    underfill = jnp.maximum(area_t - area_p, 0.0) / (area_t + 1e-7)
    area_term = rel_err + (UNDERFILL_FACTOR - 1.0) * underfill
    centroid_term = jnp.abs(cp_num / cp_den - ct_num / ct_den)
    depth_term = dsum / (nf + 1.0)
    valid = vld_b & (e > s) & jnp.logical_not(area_t < MIN_AREA)
    cnt = jnp.sum(jnp.where(valid, 1.0, 0.0))
    a = jnp.sum(jnp.where(valid, area_term, 0.0))
    cc = jnp.sum(jnp.where(valid, centroid_term, 0.0))
    dd = jnp.sum(jnp.where(valid, depth_term, 0.0))
    den = jnp.maximum(cnt, 1.0)
    num = W_AREA * a + W_CENTROID * cc + W_DEPTH * dd
    loss = jnp.full((1, 1), num) / jnp.full((1, 1), den)
    loss = jnp.where(jnp.full((1, 1), cnt) > 0.0, loss,
                     jnp.zeros((1, 1), jnp.float32))
    out_ref[:] = loss


def _terms(tv, pv, c8):
    return pl.pallas_call(
        _terms_body,
        out_shape=jax.ShapeDtypeStruct((1, 1), jnp.float32),
    )(tv, pv, c8)


def kernel(prediction, target, lam_nm):
    del lam_nm
    pred = prediction.astype(jnp.float32)
    tgt = target.astype(jnp.float32)
    centers_enc = _detect(tgt)
    tv, pv = _sc_gather()(tgt, pred, centers_enc)
    loss = _terms(tv, pv, centers_enc)
    return loss.reshape(())


# speedup vs baseline: 4.7509x; 1.0119x over previous
import functools

import jax
import jax.numpy as jnp
import numpy as np
from jax import lax
from jax.experimental import pallas as pl
from jax.experimental.pallas import tpu as pltpu
from jax.experimental.pallas import tpu_sc as plsc

ROI_LO_I, ROI_HI_I = 40, 400
M_DIPS = 6
MIN_AREA = 1e-05
W_AREA = 1.0
W_CENTROID = 1.0
W_DEPTH = 0.2
UNDERFILL_FACTOR = 2.0
B, L = 64, 2048
HALF = 10
WN = 2 * HALF + 1
NEG = float("-inf")

NSUB = 16
NCORE = 2
NW = NCORE * NSUB
ROWS_PER = B // NW


def _log_taps():
    sigma = 2.0
    radius = int(max(1.0, 3.0 * sigma))
    x = np.arange(-radius, radius + 1, dtype=np.float32)
    s2 = np.float32(sigma * sigma)
    g = np.exp(-(x ** 2) / (2.0 * s2)).astype(np.float32)
    taps = ((x ** 2 - s2) / s2 ** 2 * g).astype(np.float32)
    taps = (taps - taps.mean()).astype(np.float32)
    return taps


_TAPS = _log_taps()

BK0 = 32
BW = 512


def _detect_body(t_ref, centers_ref, pool_ref):
    acc = jnp.zeros((B, BW), jnp.float32)
    for k in range(_TAPS.shape[0]):
        off = BK0 - 6 + k
        acc = acc + float(_TAPS[k]) * t_ref[:, off:off + BW]
    colb = lax.broadcasted_iota(jnp.int32, (B, BW), 1)
    roi = ((colb >= ROI_LO_I - BK0) & (colb <= ROI_HI_I - BK0)).astype(
        jnp.float32)
    scores = -acc * roi

    pool_ref[:] = jnp.zeros((B, BW + 128), jnp.float32)
    pool_ref[:, 64:64 + BW] = scores
    pooled = pool_ref[:, 59:59 + BW]
    for d in range(1, 11):
        pooled = jnp.maximum(pooled, pool_ref[:, 59 + d:59 + d + BW])

    mean = jnp.sum(scores, axis=1, keepdims=True) * (1.0 / L)
    keep = (scores == pooled) & (scores > mean)
    masked = jnp.where(keep & (scores > 0.0), scores, NEG)

    cbs = []
    pvs = []
    for tk in range(M_DIPS):
        m = jnp.max(masked, axis=1, keepdims=True)
        ismax = masked == m
        cb = jnp.min(jnp.where(ismax, colb, BW), axis=1, keepdims=True)
        cbs.append(cb)
        pvs.append(m > NEG)
        masked = jnp.where(colb == cb, NEG, masked)

    p_cnt = jnp.zeros((B, 1), jnp.int32)
    for pv in pvs:
        p_cnt = p_cnt + pv.astype(jnp.int32)
    neg_mean = mean < 0.0

    col16 = lax.broadcasted_iota(jnp.int32, (B, 16), 1)
    centers16 = jnp.full((B, 16), -1, jnp.int32)
    for tk in range(M_DIPS):
        c = jnp.where(pvs[tk], BK0 + cbs[tk], tk - p_cnt)
        enc = jnp.where(pvs[tk] | neg_mean, c, -1)
        centers16 = jnp.where(col16 == tk, enc, centers16)
    centers_ref[:] = centers16


def _detect(target):
    return pl.pallas_call(
        _detect_body,
        out_shape=jax.ShapeDtypeStruct((B, 16), jnp.int32),
        scratch_shapes=[
            pltpu.VMEM((B, BW + 128), jnp.float32),
        ],
    )(target)


def _sc_gather_body(tgt_hbm, pred_hbm, cenc_hbm, tv_hbm, pv_hbm,
                    trows, prows, crows, twin, pwin, sem):
    wid = lax.axis_index("s") * NCORE + lax.axis_index("c")
    base = wid * ROWS_PER
    copies = [
        pltpu.async_copy(cenc_hbm.at[pl.ds(base, ROWS_PER)], crows, sem),
        pltpu.async_copy(tgt_hbm.at[pl.ds(base, ROWS_PER)], trows, sem),
        pltpu.async_copy(pred_hbm.at[pl.ds(base, ROWS_PER)], prows, sem),
    ]
    for cp in copies:
        cp.wait()
    for r in range(ROWS_PER):
        ce = crows[r]
        c = jnp.maximum(ce, 0)
        s = jnp.maximum(c - HALF, 0)
        e = jnp.minimum(c + HALF, L - 1)
        rvec = jnp.full((16,), r, jnp.int32)

        def jbody(j, carry, r=r, s=s, e=e, rvec=rvec):
            idx = jnp.minimum(s + j, e)
            twin[j, r] = plsc.load_gather(trows, [rvec, idx])
            pwin[j, r] = plsc.load_gather(prows, [rvec, idx])
            return carry

        lax.fori_loop(0, WN, jbody, 0)
    outs = [
        pltpu.async_copy(twin, tv_hbm.at[:, pl.ds(base, ROWS_PER)], sem),
        pltpu.async_copy(pwin, pv_hbm.at[:, pl.ds(base, ROWS_PER)], sem),
    ]
    for cp in outs:
        cp.wait()


@functools.cache
def _sc_gather():
  return pl.kernel(
    _sc_gather_body,
    out_type=[
        jax.ShapeDtypeStruct((WN, B, 16), jnp.float32),
        jax.ShapeDtypeStruct((WN, B, 16), jnp.float32),
    ],
    mesh=plsc.VectorSubcoreMesh(core_axis_name="c", subcore_axis_name="s",
                                num_cores=NCORE, num_subcores=NSUB),
    compiler_params=pltpu.CompilerParams(needs_layout_passes=False),
    scratch_types=[
        pltpu.VMEM((ROWS_PER, L), jnp.float32),
        pltpu.VMEM((ROWS_PER, L), jnp.float32),
        pltpu.VMEM((ROWS_PER, 16), jnp.int32),
        pltpu.VMEM((WN, ROWS_PER, 16), jnp.float32),
        pltpu.VMEM((WN, ROWS_PER, 16), jnp.float32),
        pltpu.SemaphoreType.DMA,
    ],
  )


def _terms_body(tv_ref, pv_ref, c8_ref, out_ref):
    ce = c8_ref[:]
    vld_b = ce >= 0
    c = jnp.where(vld_b, ce, 0)
    s = jnp.maximum(c - HALF, 0)
    e = jnp.minimum(c + HALF, L - 1)
    n = e - s
    nf = n.astype(jnp.float32)
    lam_s = 300.0 + 0.5 * s.astype(jnp.float32)
    lam_e = 300.0 + 0.5 * e.astype(jnp.float32)
    dlam = lam_e - lam_s + 1e-6
    ts = tv_ref[0]
    te = tv_ref[WN - 1]
    ps = pv_ref[0]
    pe = pv_ref[WN - 1]
    zero = jnp.zeros((B, 16), jnp.float32)
    area_t = zero
    area_p = zero
    ct_num = zero
    ct_den = zero
    cp_num = zero
    cp_den = zero
    dsum = zero
    prev_dt = zero
    prev_dp = zero
    prev_lseg = zero
    for j in range(WN):
        idx = jnp.minimum(s + j, e)
        lseg = 300.0 + 0.5 * idx.astype(jnp.float32)
        tt = (lseg - lam_s) / dlam
        cont_t = jnp.maximum((1.0 - tt) * ts + tt * te, 1e-6)
        cont_p = jnp.maximum((1.0 - tt) * ps + tt * pe, 1e-6)
        tv = tv_ref[j]
        pv = pv_ref[j]
        dt = jnp.clip(1.0 - jnp.clip(tv / cont_t, 0.0, 2.0), 0.0, 1.0)
        dp = jnp.clip(1.0 - jnp.clip(pv / cont_p, 0.0, 2.0), 0.0, 1.0)
        jf = float(j)
        pm = jf <= nf
        if j > 0:
            sm = (jf - 1.0) < nf
            dl = lseg - prev_lseg
            area_t = area_t + jnp.where(sm, (dt + prev_dt) * 0.5 * dl, 0.0)
            area_p = area_p + jnp.where(sm, (dp + prev_dp) * 0.5 * dl, 0.0)
        wt = dt + 1e-7
        wp = dp + 1e-7
        ct_num = ct_num + jnp.where(pm, lseg * wt, 0.0)
        ct_den = ct_den + jnp.where(pm, wt, 0.0)
        cp_num = cp_num + jnp.where(pm, lseg * wp, 0.0)
        cp_den = cp_den + jnp.where(pm, wp, 0.0)
        dsum = dsum + jnp.where(pm, jnp.abs(dp - dt), 0.0)
        prev_dt = dt
        prev_dp = dp
        prev_lseg = lseg
    rel_err = jnp.abs(area_p - area_t) / (area_t + 1e-7)
    underfill = jnp.maximum(area_t - area_p, 0.0) / (area_t + 1e-7)
    area_term = rel_err + (UNDERFILL_FACTOR - 1.0) * underfill
    centroid_term = jnp.abs(cp_num / cp_den - ct_num / ct_den)
    depth_term = dsum / (nf + 1.0)
    valid = vld_b & (e > s) & jnp.logical_not(area_t < MIN_AREA)
    cnt = jnp.sum(jnp.where(valid, 1.0, 0.0))
    a = jnp.sum(jnp.where(valid, area_term, 0.0))
    cc = jnp.sum(jnp.where(valid, centroid_term, 0.0))
    dd = jnp.sum(jnp.where(valid, depth_term, 0.0))
    den = jnp.maximum(cnt, 1.0)
    num = W_AREA * a + W_CENTROID * cc + W_DEPTH * dd
    loss = jnp.full((1, 1), num) / jnp.full((1, 1), den)
    loss = jnp.where(jnp.full((1, 1), cnt) > 0.0, loss,
                     jnp.zeros((1, 1), jnp.float32))
    out_ref[:] = loss


def _terms(tv, pv, c8):
    return pl.pallas_call(
        _terms_body,
        out_shape=jax.ShapeDtypeStruct((1, 1), jnp.float32),
    )(tv, pv, c8)


def kernel(prediction, target, lam_nm):
    del lam_nm
    pred = prediction.astype(jnp.float32)
    tgt = target.astype(jnp.float32)
    centers_enc = _detect(tgt)
    tv, pv = _sc_gather()(tgt, pred, centers_enc)
    loss = _terms(tv, pv, centers_enc)
    return loss.reshape(())
